# baseline (speedup 1.0000x reference)
import functools

import jax
import jax.numpy as jnp
from jax import lax
from jax.experimental import pallas as pl
from jax.experimental.pallas import tpu as pltpu
from jax.experimental.pallas import tpu_sc as plsc

_TILE = 256
_N_WORKERS = 32


def _route_body(x_ref, Wg_ref, bg_ref, w1_ref, w2_ref, pos_ref, te_ref,
                *, T, E, n_tiles):
    x = x_ref[...]
    logits = lax.dot_general(x, Wg_ref[...], (((1,), (1,)), ((), ())),
                             preferred_element_type=jnp.float32) + bg_ref[0, :]
    m = jnp.max(logits, axis=-1, keepdims=True)
    p = jnp.exp(logits - m)
    p = p / jnp.sum(p, axis=-1, keepdims=True)
    cols = lax.broadcasted_iota(jnp.int32, p.shape, 1)
    idx1 = jnp.argmax(p, axis=-1)[:, None]
    mask1 = cols == idx1
    p1 = jnp.sum(jnp.where(mask1, p, 0.0), axis=-1, keepdims=True)
    p_m = jnp.where(mask1, -jnp.inf, p)
    idx2 = jnp.argmax(p_m, axis=-1)[:, None]
    mask2 = cols == idx2
    p2 = jnp.sum(jnp.where(mask2, p, 0.0), axis=-1, keepdims=True)
    e2v = jnp.exp(p2 - p1)
    s = 1.0 + e2v
    w1_ref[...] = 1.0 / s
    w2_ref[...] = e2v / s

    R = T // 128
    rb = lax.broadcasted_iota(jnp.int32, (T, R), 0) // 128
    bb = lax.broadcasted_iota(jnp.int32, (T, R), 1)
    S = (rb == bb).astype(jnp.float32)
    rc = lax.broadcasted_iota(jnp.int32, (T, 128), 0) % 128
    cc = lax.broadcasted_iota(jnp.int32, (T, 128), 1)
    C = (rc == cc).astype(jnp.float32)

    def resh(v):
        return lax.dot_general(S * v, C, (((0,), (0,)), ((), ())),
                               preferred_element_type=jnp.float32)

    e1f = resh(idx1.astype(jnp.float32))
    e2f = resh(idx2.astype(jnp.float32))

    r1 = lax.broadcasted_iota(jnp.int32, (128, 128), 0)
    c1 = lax.broadcasted_iota(jnp.int32, (128, 128), 1)
    SU = (r1 < c1).astype(jnp.float32)
    rL = lax.broadcasted_iota(jnp.int32, (R, R), 0)
    cL = lax.broadcasted_iota(jnp.int32, (R, R), 1)
    LS = (rL > cL).astype(jnp.float32)

    def prefix(me):
        within = lax.dot_general(me, SU, (((1,), (0,)), ((), ())),
                                 preferred_element_type=jnp.float32)
        rows = jnp.sum(me, axis=1, keepdims=True)
        off = lax.dot_general(LS, rows, (((1,), (0,)), ((), ())),
                              preferred_element_type=jnp.float32)
        return within + off, jnp.sum(rows)

    pos1 = jnp.zeros((R, 128), jnp.float32)
    pos2 = jnp.zeros((R, 128), jnp.float32)
    te = jnp.zeros((1, 128), jnp.int32)
    tidv = lax.broadcasted_iota(jnp.int32, (1, 128), 1) * _TILE
    start = jnp.int32(0)
    for e in range(E):
        m1e = (e1f == e).astype(jnp.float32)
        m2e = (e2f == e).astype(jnp.float32)
        rank1, cnt1 = prefix(m1e)
        rank2, cnt2 = prefix(m2e)
        startf = start.astype(jnp.float32)
        pos1 += m1e * (startf + rank1)
        pos2 += m2e * (startf + cnt1 + rank2)
        cnt = (cnt1 + cnt2).astype(jnp.int32)
        padded = ((cnt + _TILE - 1) // _TILE) * _TILE
        start = start + padded
        te += (tidv >= start).astype(jnp.int32)
    pos_ref[0] = pos1.astype(jnp.int32)
    pos_ref[1] = pos2.astype(jnp.int32)
    te_ref[...] = jnp.minimum(te, E - 1)


def _mlp_body(te_ref, xs_ref, W1_ref, b1_ref, W2_ref, b2_ref, buf_ref):
    xb = xs_ref[...].astype(jnp.bfloat16)
    h = lax.dot_general(xb, W1_ref[0], (((1,), (1,)), ((), ())),
                        preferred_element_type=jnp.float32) + b1_ref[0, 0, :]
    h = jnp.maximum(h, 0.0).astype(jnp.bfloat16)
    buf_ref[...] = lax.dot_general(h, W2_ref[0], (((1,), (1,)), ((), ())),
                                   preferred_element_type=jnp.float32) \
        + b2_ref[0, 0, :]


def _final_body(x_ref, g1_ref, g2_ref, w1_ref, w2_ref, gamma_ref, beta_ref,
                out_ref):
    y = x_ref[...] + w1_ref[...] * g1_ref[...] + w2_ref[...] * g2_ref[...]
    mu = jnp.mean(y, axis=-1, keepdims=True)
    yc = y - mu
    var = jnp.mean(yc * yc, axis=-1, keepdims=True)
    out_ref[...] = yc * lax.rsqrt(var + 1e-5) * gamma_ref[0, :] + beta_ref[0, :]


def _sc_dispatch(x, pos3, *, n_rows, n_chunk):
    T, D = x.shape
    mesh = plsc.VectorSubcoreMesh(core_axis_name="c", subcore_axis_name="s")

    @functools.partial(
        pl.kernel,
        out_type=jax.ShapeDtypeStruct((n_rows, D), jnp.float32),
        mesh=mesh,
        scratch_types=[
            pltpu.VMEM((n_chunk // 64, 64), jnp.int32),
            pltpu.VMEM((64, D), jnp.float32),
            pltpu.SemaphoreType.DMA,
        ],
    )
    def dispatch(x_hbm, pos_hbm, xs_hbm, idx_v, rows_v, sem):
        wid = lax.axis_index("s") * 2 + lax.axis_index("c")
        base = wid * n_chunk
        tok_base = base % T
        pltpu.sync_copy(pos_hbm.at[wid], idx_v)
        for j in range(n_chunk // 64):
            pltpu.sync_copy(x_hbm.at[pl.ds(tok_base + j * 64, 64)], rows_v)
            for i in range(4):
                idx16 = idx_v[j, pl.ds(i * 16, 16)]
                pltpu.async_copy(rows_v.at[pl.ds(i * 16, 16)],
                                 xs_hbm.at[idx16], sem).wait()

    return dispatch(x, pos3)


def _sc_combine(buf, pos3, *, n_pairs, n_chunk):
    D = buf.shape[1]
    mesh = plsc.VectorSubcoreMesh(core_axis_name="c", subcore_axis_name="s")

    @functools.partial(
        pl.kernel,
        out_type=jax.ShapeDtypeStruct((n_pairs, D), jnp.float32),
        mesh=mesh,
        scratch_types=[
            pltpu.VMEM((n_chunk // 64, 64), jnp.int32),
            pltpu.VMEM((64, D), jnp.float32),
            pltpu.SemaphoreType.DMA,
        ],
    )
    def combine(buf_hbm, pos_hbm, g_hbm, idx_v, rows_v, sem):
        wid = lax.axis_index("s") * 2 + lax.axis_index("c")
        base = wid * n_chunk
        pltpu.sync_copy(pos_hbm.at[wid], idx_v)
        for j in range(n_chunk // 64):
            pltpu.async_copy(buf_hbm.at[idx_v.at[j]], rows_v, sem).wait()
            pltpu.sync_copy(rows_v, g_hbm.at[pl.ds(base + j * 64, 64)])

    return combine(buf, pos3)


def kernel(x, Wg, bg, W1, b1, W2, b2, gamma, beta):
    T, D = x.shape
    E, H, _ = W1.shape
    n_pairs = 2 * T
    n_tiles = (n_pairs + E * _TILE) // _TILE
    n_rows = n_tiles * _TILE
    R = T // 128

    route = pl.pallas_call(
        functools.partial(_route_body, T=T, E=E, n_tiles=n_tiles),
        grid=(1,),
        in_specs=[
            pl.BlockSpec((T, D), lambda i: (0, 0)),
            pl.BlockSpec((E, D), lambda i: (0, 0)),
            pl.BlockSpec((1, E), lambda i: (0, 0)),
        ],
        out_specs=[
            pl.BlockSpec((T, 1), lambda i: (0, 0)),
            pl.BlockSpec((T, 1), lambda i: (0, 0)),
            pl.BlockSpec((2, R, 128), lambda i: (0, 0, 0)),
            pl.BlockSpec((1, 128), lambda i: (0, 0)),
        ],
        out_shape=[
            jax.ShapeDtypeStruct((T, 1), jnp.float32),
            jax.ShapeDtypeStruct((T, 1), jnp.float32),
            jax.ShapeDtypeStruct((2, R, 128), jnp.int32),
            jax.ShapeDtypeStruct((1, 128), jnp.int32),
        ],
        compiler_params=pltpu.CompilerParams(
            dimension_semantics=("arbitrary",),
        ),
    )(x, Wg, bg.reshape(1, E))
    w1c, w2c, pos, te = route
    n_chunk = n_pairs // _N_WORKERS
    pos3 = pos.reshape(_N_WORKERS, n_chunk // 64, 64)
    te_flat = te.reshape(128)[:n_tiles]

    x_sorted = _sc_dispatch(x, pos3, n_rows=n_rows, n_chunk=n_chunk)

    grid_spec = pltpu.PrefetchScalarGridSpec(
        num_scalar_prefetch=1,
        grid=(n_tiles,),
        in_specs=[
            pl.BlockSpec((_TILE, D), lambda i, te_s: (i, 0)),
            pl.BlockSpec((1, H, D), lambda i, te_s: (te_s[i], 0, 0)),
            pl.BlockSpec((1, 1, H), lambda i, te_s: (te_s[i], 0, 0)),
            pl.BlockSpec((1, D, H), lambda i, te_s: (te_s[i], 0, 0)),
            pl.BlockSpec((1, 1, D), lambda i, te_s: (te_s[i], 0, 0)),
        ],
        out_specs=pl.BlockSpec((_TILE, D), lambda i, te_s: (i, 0)),
    )
    buf = pl.pallas_call(
        _mlp_body,
        grid_spec=grid_spec,
        out_shape=jax.ShapeDtypeStruct((n_rows, D), jnp.float32),
        compiler_params=pltpu.CompilerParams(
            dimension_semantics=("arbitrary",),
        ),
    )(te_flat, x_sorted, W1.astype(jnp.bfloat16), b1.reshape(E, 1, H),
      W2.astype(jnp.bfloat16), b2.reshape(E, 1, D))

    g = _sc_combine(buf, pos3, n_pairs=n_pairs, n_chunk=n_chunk)
    g1 = g[:T]
    g2 = g[T:]

    tile_t = 256
    out = pl.pallas_call(
        _final_body,
        grid=(T // tile_t,),
        in_specs=[
            pl.BlockSpec((tile_t, D), lambda i: (i, 0)),
            pl.BlockSpec((tile_t, D), lambda i: (i, 0)),
            pl.BlockSpec((tile_t, D), lambda i: (i, 0)),
            pl.BlockSpec((tile_t, 1), lambda i: (i, 0)),
            pl.BlockSpec((tile_t, 1), lambda i: (i, 0)),
            pl.BlockSpec((1, D), lambda i: (0, 0)),
            pl.BlockSpec((1, D), lambda i: (0, 0)),
        ],
        out_specs=pl.BlockSpec((tile_t, D), lambda i: (i, 0)),
        out_shape=jax.ShapeDtypeStruct((T, D), jnp.float32),
        compiler_params=pltpu.CompilerParams(
            dimension_semantics=("arbitrary",),
        ),
    )(x, g1, g2, w1c, w2c, gamma.reshape(1, D), beta.reshape(1, D))
    return out


# speedup vs baseline: 1.1888x; 1.1888x over previous
import functools

import jax
import jax.numpy as jnp
from jax import lax
from jax.experimental import pallas as pl
from jax.experimental.pallas import tpu as pltpu
from jax.experimental.pallas import tpu_sc as plsc

_TILE = 256
_N_WORKERS = 32


def _route_body(x_ref, Wg_ref, bg_ref, w1_ref, w2_ref, pos_ref, te_ref,
                *, T, E, n_tiles):
    x = x_ref[...]
    logits = lax.dot_general(x, Wg_ref[...], (((1,), (1,)), ((), ())),
                             preferred_element_type=jnp.float32) + bg_ref[0, :]
    m = jnp.max(logits, axis=-1, keepdims=True)
    p = jnp.exp(logits - m)
    p = p / jnp.sum(p, axis=-1, keepdims=True)
    cols = lax.broadcasted_iota(jnp.int32, p.shape, 1)
    idx1 = jnp.argmax(p, axis=-1)[:, None]
    mask1 = cols == idx1
    p1 = jnp.sum(jnp.where(mask1, p, 0.0), axis=-1, keepdims=True)
    p_m = jnp.where(mask1, -jnp.inf, p)
    idx2 = jnp.argmax(p_m, axis=-1)[:, None]
    mask2 = cols == idx2
    p2 = jnp.sum(jnp.where(mask2, p, 0.0), axis=-1, keepdims=True)
    e2v = jnp.exp(p2 - p1)
    s = 1.0 + e2v
    w1_ref[...] = 1.0 / s
    w2_ref[...] = e2v / s

    R = T // 128
    rb = lax.broadcasted_iota(jnp.int32, (T, R), 0) // 128
    bb = lax.broadcasted_iota(jnp.int32, (T, R), 1)
    S = (rb == bb).astype(jnp.float32)
    rc = lax.broadcasted_iota(jnp.int32, (T, 128), 0) % 128
    cc = lax.broadcasted_iota(jnp.int32, (T, 128), 1)
    C = (rc == cc).astype(jnp.float32)

    def resh(v):
        return lax.dot_general(S * v, C, (((0,), (0,)), ((), ())),
                               preferred_element_type=jnp.float32)

    e1f = resh(idx1.astype(jnp.float32))
    e2f = resh(idx2.astype(jnp.float32))

    r1 = lax.broadcasted_iota(jnp.int32, (128, 128), 0)
    c1 = lax.broadcasted_iota(jnp.int32, (128, 128), 1)
    SU = (r1 < c1).astype(jnp.float32)
    rL = lax.broadcasted_iota(jnp.int32, (R, R), 0)
    cL = lax.broadcasted_iota(jnp.int32, (R, R), 1)
    LS = (rL > cL).astype(jnp.float32)

    def prefix(me):
        within = lax.dot_general(me, SU, (((1,), (0,)), ((), ())),
                                 preferred_element_type=jnp.float32)
        rows = jnp.sum(me, axis=1, keepdims=True)
        off = lax.dot_general(LS, rows, (((1,), (0,)), ((), ())),
                              preferred_element_type=jnp.float32)
        return within + off, jnp.sum(rows)

    pos1 = jnp.zeros((R, 128), jnp.float32)
    pos2 = jnp.zeros((R, 128), jnp.float32)
    te = jnp.zeros((1, 128), jnp.int32)
    tidv = lax.broadcasted_iota(jnp.int32, (1, 128), 1) * _TILE
    start = jnp.int32(0)
    for e in range(E):
        m1e = (e1f == e).astype(jnp.float32)
        m2e = (e2f == e).astype(jnp.float32)
        rank1, cnt1 = prefix(m1e)
        rank2, cnt2 = prefix(m2e)
        startf = start.astype(jnp.float32)
        pos1 += m1e * (startf + rank1)
        pos2 += m2e * (startf + cnt1 + rank2)
        cnt = (cnt1 + cnt2).astype(jnp.int32)
        padded = ((cnt + _TILE - 1) // _TILE) * _TILE
        start = start + padded
        te += (tidv >= start).astype(jnp.int32)
    pos_ref[0] = pos1.astype(jnp.int32)
    pos_ref[1] = pos2.astype(jnp.int32)
    te_ref[...] = jnp.minimum(te, E - 1)


def _mlp_body(te_ref, xs_ref, W1_ref, b1_ref, W2_ref, b2_ref, buf_ref):
    h = lax.dot_general(xs_ref[...], W1_ref[0], (((1,), (1,)), ((), ())),
                        preferred_element_type=jnp.float32) + b1_ref[0, 0, :]
    h = jnp.maximum(h, 0.0)
    buf_ref[...] = lax.dot_general(h, W2_ref[0], (((1,), (1,)), ((), ())),
                                   preferred_element_type=jnp.float32) \
        + b2_ref[0, 0, :]


def _final_body(x_ref, g1_ref, g2_ref, w1_ref, w2_ref, gamma_ref, beta_ref,
                out_ref):
    y = x_ref[...] + w1_ref[...] * g1_ref[...] + w2_ref[...] * g2_ref[...]
    mu = jnp.mean(y, axis=-1, keepdims=True)
    yc = y - mu
    var = jnp.mean(yc * yc, axis=-1, keepdims=True)
    out_ref[...] = yc * lax.rsqrt(var + 1e-5) * gamma_ref[0, :] + beta_ref[0, :]


def _sc_dispatch(x, pos3, *, n_rows, n_chunk):
    T, D = x.shape
    mesh = plsc.VectorSubcoreMesh(core_axis_name="c", subcore_axis_name="s")

    @functools.partial(
        pl.kernel,
        out_type=jax.ShapeDtypeStruct((n_rows, D), jnp.float32),
        mesh=mesh,
        scratch_types=[
            pltpu.VMEM((n_chunk // 64, 64), jnp.int32),
            pltpu.VMEM((64, D), jnp.float32),
            pltpu.VMEM((64, D), jnp.float32),
            pltpu.SemaphoreType.DMA,
            pltpu.SemaphoreType.DMA,
        ],
    )
    def dispatch(x_hbm, pos_hbm, xs_hbm, idx_v, rows_a, rows_b, sem_l, sem_s):
        wid = lax.axis_index("s") * 2 + lax.axis_index("c")
        base = wid * n_chunk
        tok_base = base % T
        pltpu.sync_copy(pos_hbm.at[wid], idx_v)
        nj = n_chunk // 64
        bufs = (rows_a, rows_b)
        ld = pltpu.async_copy(x_hbm.at[pl.ds(tok_base, 64)], rows_a, sem_l)
        for j in range(nj):
            cur = bufs[j % 2]
            ld.wait()
            if j + 1 < nj:
                ld = pltpu.async_copy(
                    x_hbm.at[pl.ds(tok_base + (j + 1) * 64, 64)],
                    bufs[(j + 1) % 2], sem_l)
            descs = []
            for i in range(4):
                idx16 = idx_v[j, pl.ds(i * 16, 16)]
                descs.append(pltpu.async_copy(cur.at[pl.ds(i * 16, 16)],
                                              xs_hbm.at[idx16], sem_s))
            for dsc in descs:
                dsc.wait()

    return dispatch(x, pos3)


def _sc_combine(buf, pos3, *, n_pairs, n_chunk):
    D = buf.shape[1]
    mesh = plsc.VectorSubcoreMesh(core_axis_name="c", subcore_axis_name="s")

    @functools.partial(
        pl.kernel,
        out_type=jax.ShapeDtypeStruct((n_pairs, D), jnp.float32),
        mesh=mesh,
        scratch_types=[
            pltpu.VMEM((n_chunk // 64, 64), jnp.int32),
            pltpu.VMEM((64, D), jnp.float32),
            pltpu.VMEM((64, D), jnp.float32),
            pltpu.SemaphoreType.DMA,
        ],
    )
    def combine(buf_hbm, pos_hbm, g_hbm, idx_v, rows_a, rows_b, sem_g):
        wid = lax.axis_index("s") * 2 + lax.axis_index("c")
        base = wid * n_chunk
        pltpu.sync_copy(pos_hbm.at[wid], idx_v)
        nj = n_chunk // 64
        bufs = (rows_a, rows_b)
        gd = pltpu.async_copy(buf_hbm.at[idx_v.at[0]], rows_a, sem_g)
        for j in range(nj):
            cur = bufs[j % 2]
            gd.wait()
            if j + 1 < nj:
                gd = pltpu.async_copy(buf_hbm.at[idx_v.at[j + 1]],
                                      bufs[(j + 1) % 2], sem_g)
            pltpu.sync_copy(cur, g_hbm.at[pl.ds(base + j * 64, 64)])

    return combine(buf, pos3)


def kernel(x, Wg, bg, W1, b1, W2, b2, gamma, beta):
    T, D = x.shape
    E, H, _ = W1.shape
    n_pairs = 2 * T
    n_tiles = (n_pairs + E * _TILE) // _TILE
    n_rows = n_tiles * _TILE
    R = T // 128

    route = pl.pallas_call(
        functools.partial(_route_body, T=T, E=E, n_tiles=n_tiles),
        grid=(1,),
        in_specs=[
            pl.BlockSpec((T, D), lambda i: (0, 0)),
            pl.BlockSpec((E, D), lambda i: (0, 0)),
            pl.BlockSpec((1, E), lambda i: (0, 0)),
        ],
        out_specs=[
            pl.BlockSpec((T, 1), lambda i: (0, 0)),
            pl.BlockSpec((T, 1), lambda i: (0, 0)),
            pl.BlockSpec((2, R, 128), lambda i: (0, 0, 0)),
            pl.BlockSpec((1, 128), lambda i: (0, 0)),
        ],
        out_shape=[
            jax.ShapeDtypeStruct((T, 1), jnp.float32),
            jax.ShapeDtypeStruct((T, 1), jnp.float32),
            jax.ShapeDtypeStruct((2, R, 128), jnp.int32),
            jax.ShapeDtypeStruct((1, 128), jnp.int32),
        ],
        compiler_params=pltpu.CompilerParams(
            dimension_semantics=("arbitrary",),
        ),
    )(x, Wg, bg.reshape(1, E))
    w1c, w2c, pos, te = route
    n_chunk = n_pairs // _N_WORKERS
    pos3 = pos.reshape(_N_WORKERS, n_chunk // 64, 64)
    te_flat = te.reshape(128)[:n_tiles]

    x_sorted = _sc_dispatch(x, pos3, n_rows=n_rows, n_chunk=n_chunk)

    grid_spec = pltpu.PrefetchScalarGridSpec(
        num_scalar_prefetch=1,
        grid=(n_tiles,),
        in_specs=[
            pl.BlockSpec((_TILE, D), lambda i, te_s: (i, 0)),
            pl.BlockSpec((1, H, D), lambda i, te_s: (te_s[i], 0, 0)),
            pl.BlockSpec((1, 1, H), lambda i, te_s: (te_s[i], 0, 0)),
            pl.BlockSpec((1, D, H), lambda i, te_s: (te_s[i], 0, 0)),
            pl.BlockSpec((1, 1, D), lambda i, te_s: (te_s[i], 0, 0)),
        ],
        out_specs=pl.BlockSpec((_TILE, D), lambda i, te_s: (i, 0)),
    )
    buf = pl.pallas_call(
        _mlp_body,
        grid_spec=grid_spec,
        out_shape=jax.ShapeDtypeStruct((n_rows, D), jnp.float32),
        compiler_params=pltpu.CompilerParams(
            dimension_semantics=("arbitrary",),
        ),
    )(te_flat, x_sorted, W1, b1.reshape(E, 1, H), W2, b2.reshape(E, 1, D))

    g = _sc_combine(buf, pos3, n_pairs=n_pairs, n_chunk=n_chunk)

    tile_t = 256
    half = T // tile_t
    out = pl.pallas_call(
        _final_body,
        grid=(T // tile_t,),
        in_specs=[
            pl.BlockSpec((tile_t, D), lambda i: (i, 0)),
            pl.BlockSpec((tile_t, D), lambda i: (i, 0)),
            pl.BlockSpec((tile_t, D), lambda i: (i + half, 0)),
            pl.BlockSpec((tile_t, 1), lambda i: (i, 0)),
            pl.BlockSpec((tile_t, 1), lambda i: (i, 0)),
            pl.BlockSpec((1, D), lambda i: (0, 0)),
            pl.BlockSpec((1, D), lambda i: (0, 0)),
        ],
        out_specs=pl.BlockSpec((tile_t, D), lambda i: (i, 0)),
        out_shape=jax.ShapeDtypeStruct((T, D), jnp.float32),
        compiler_params=pltpu.CompilerParams(
            dimension_semantics=("arbitrary",),
        ),
    )(x, g, g, w1c, w2c, gamma.reshape(1, D), beta.reshape(1, D))
    return out


# speedup vs baseline: 1.2163x; 1.0231x over previous
import functools

import jax
import jax.numpy as jnp
from jax import lax
from jax.experimental import pallas as pl
from jax.experimental.pallas import tpu as pltpu
from jax.experimental.pallas import tpu_sc as plsc

_TILE = 256
_N_WORKERS = 32


def _route_body(x_ref, Wg_ref, bg_ref, w1_ref, w2_ref, pos_ref, te_ref,
                *, T, E, n_tiles):
    x = x_ref[...]
    logits = lax.dot_general(x, Wg_ref[...], (((1,), (1,)), ((), ())),
                             preferred_element_type=jnp.float32) + bg_ref[0, :]
    m = jnp.max(logits, axis=-1, keepdims=True)
    p = jnp.exp(logits - m)
    p = p / jnp.sum(p, axis=-1, keepdims=True)
    cols = lax.broadcasted_iota(jnp.int32, p.shape, 1)
    idx1 = jnp.argmax(p, axis=-1)[:, None]
    mask1 = cols == idx1
    p1 = jnp.sum(jnp.where(mask1, p, 0.0), axis=-1, keepdims=True)
    p_m = jnp.where(mask1, -jnp.inf, p)
    idx2 = jnp.argmax(p_m, axis=-1)[:, None]
    mask2 = cols == idx2
    p2 = jnp.sum(jnp.where(mask2, p, 0.0), axis=-1, keepdims=True)
    e2v = jnp.exp(p2 - p1)
    s = 1.0 + e2v
    w1_ref[...] = 1.0 / s
    w2_ref[...] = e2v / s

    R = T // 128
    rb = lax.broadcasted_iota(jnp.int32, (T, R), 0) // 128
    bb = lax.broadcasted_iota(jnp.int32, (T, R), 1)
    S = (rb == bb).astype(jnp.float32)
    rc = lax.broadcasted_iota(jnp.int32, (T, 128), 0) % 128
    cc = lax.broadcasted_iota(jnp.int32, (T, 128), 1)
    C = (rc == cc).astype(jnp.float32)

    def resh(v):
        return lax.dot_general(S * v, C, (((0,), (0,)), ((), ())),
                               preferred_element_type=jnp.float32)

    e1f = resh(idx1.astype(jnp.float32))
    e2f = resh(idx2.astype(jnp.float32))

    r1 = lax.broadcasted_iota(jnp.int32, (128, 128), 0)
    c1 = lax.broadcasted_iota(jnp.int32, (128, 128), 1)
    SU = (r1 < c1).astype(jnp.float32)
    rL = lax.broadcasted_iota(jnp.int32, (R, R), 0)
    cL = lax.broadcasted_iota(jnp.int32, (R, R), 1)
    LS = (rL > cL).astype(jnp.float32)

    def prefix(me):
        within = lax.dot_general(me, SU, (((1,), (0,)), ((), ())),
                                 preferred_element_type=jnp.float32)
        rows = jnp.sum(me, axis=1, keepdims=True)
        off = lax.dot_general(LS, rows, (((1,), (0,)), ((), ())),
                              preferred_element_type=jnp.float32)
        return within + off, jnp.sum(rows)

    pos1 = jnp.zeros((R, 128), jnp.float32)
    pos2 = jnp.zeros((R, 128), jnp.float32)
    te = jnp.zeros((1, 128), jnp.int32)
    tidv = lax.broadcasted_iota(jnp.int32, (1, 128), 1) * _TILE
    start = jnp.int32(0)
    for e in range(E):
        m1e = (e1f == e).astype(jnp.float32)
        m2e = (e2f == e).astype(jnp.float32)
        rank1, cnt1 = prefix(m1e)
        rank2, cnt2 = prefix(m2e)
        startf = start.astype(jnp.float32)
        pos1 += m1e * (startf + rank1)
        pos2 += m2e * (startf + cnt1 + rank2)
        cnt = (cnt1 + cnt2).astype(jnp.int32)
        padded = ((cnt + _TILE - 1) // _TILE) * _TILE
        start = start + padded
        te += (tidv >= start).astype(jnp.int32)
    pos_ref[0] = pos1.astype(jnp.int32)
    pos_ref[1] = pos2.astype(jnp.int32)
    col = lax.broadcasted_iota(jnp.int32, (1, 128), 1)
    te_ref[...] = jnp.where(col == 127, start // _TILE, jnp.minimum(te, E - 1))


def _mlp_body(te_ref, xs_ref, W1_ref, b1_ref, W2_ref, b2_ref, buf_ref):
    @pl.when(pl.program_id(0) < te_ref[127])
    def _():
        h = lax.dot_general(xs_ref[...], W1_ref[0], (((1,), (1,)), ((), ())),
                            preferred_element_type=jnp.float32) \
            + b1_ref[0, 0, :]
        h = jnp.maximum(h, 0.0)
        buf_ref[...] = lax.dot_general(h, W2_ref[0], (((1,), (1,)), ((), ())),
                                       preferred_element_type=jnp.float32) \
            + b2_ref[0, 0, :]


def _final_body(x_ref, g1_ref, g2_ref, w1_ref, w2_ref, gamma_ref, beta_ref,
                out_ref):
    y = x_ref[...] + w1_ref[...] * g1_ref[...] + w2_ref[...] * g2_ref[...]
    mu = jnp.mean(y, axis=-1, keepdims=True)
    yc = y - mu
    var = jnp.mean(yc * yc, axis=-1, keepdims=True)
    out_ref[...] = yc * lax.rsqrt(var + 1e-5) * gamma_ref[0, :] + beta_ref[0, :]


def _sc_dispatch(x, pos3, *, n_rows, n_chunk):
    T, D = x.shape
    mesh = plsc.VectorSubcoreMesh(core_axis_name="c", subcore_axis_name="s")

    @functools.partial(
        pl.kernel,
        out_type=jax.ShapeDtypeStruct((n_rows, D), jnp.float32),
        mesh=mesh,
        scratch_types=[
            pltpu.VMEM((n_chunk // 64, 64), jnp.int32),
            pltpu.VMEM((64, D), jnp.float32),
            pltpu.VMEM((64, D), jnp.float32),
            pltpu.SemaphoreType.DMA,
            pltpu.SemaphoreType.DMA,
        ],
    )
    def dispatch(x_hbm, pos_hbm, xs_hbm, idx_v, rows_a, rows_b, sem_l, sem_s):
        wid = lax.axis_index("s") * 2 + lax.axis_index("c")
        base = wid * n_chunk
        tok_base = base % T
        pltpu.sync_copy(pos_hbm.at[wid], idx_v)
        nj = n_chunk // 64
        bufs = (rows_a, rows_b)
        ld = pltpu.async_copy(x_hbm.at[pl.ds(tok_base, 64)], rows_a, sem_l)
        for j in range(nj):
            cur = bufs[j % 2]
            ld.wait()
            if j + 1 < nj:
                ld = pltpu.async_copy(
                    x_hbm.at[pl.ds(tok_base + (j + 1) * 64, 64)],
                    bufs[(j + 1) % 2], sem_l)
            pltpu.async_copy(cur, xs_hbm.at[idx_v.at[j]], sem_s).wait()

    return dispatch(x, pos3)


def _sc_combine(buf, pos3, *, n_pairs, n_chunk):
    D = buf.shape[1]
    mesh = plsc.VectorSubcoreMesh(core_axis_name="c", subcore_axis_name="s")

    @functools.partial(
        pl.kernel,
        out_type=jax.ShapeDtypeStruct((n_pairs, D), jnp.float32),
        mesh=mesh,
        scratch_types=[
            pltpu.VMEM((n_chunk // 64, 64), jnp.int32),
            pltpu.VMEM((64, D), jnp.float32),
            pltpu.VMEM((64, D), jnp.float32),
            pltpu.SemaphoreType.DMA,
        ],
    )
    def combine(buf_hbm, pos_hbm, g_hbm, idx_v, rows_a, rows_b, sem_g):
        wid = lax.axis_index("s") * 2 + lax.axis_index("c")
        base = wid * n_chunk
        pltpu.sync_copy(pos_hbm.at[wid], idx_v)
        nj = n_chunk // 64
        bufs = (rows_a, rows_b)
        gd = pltpu.async_copy(buf_hbm.at[idx_v.at[0]], rows_a, sem_g)
        for j in range(nj):
            cur = bufs[j % 2]
            gd.wait()
            if j + 1 < nj:
                gd = pltpu.async_copy(buf_hbm.at[idx_v.at[j + 1]],
                                      bufs[(j + 1) % 2], sem_g)
            pltpu.sync_copy(cur, g_hbm.at[pl.ds(base + j * 64, 64)])

    return combine(buf, pos3)


def kernel(x, Wg, bg, W1, b1, W2, b2, gamma, beta):
    T, D = x.shape
    E, H, _ = W1.shape
    n_pairs = 2 * T
    n_tiles = (n_pairs + E * _TILE) // _TILE
    n_rows = n_tiles * _TILE
    R = T // 128

    route = pl.pallas_call(
        functools.partial(_route_body, T=T, E=E, n_tiles=n_tiles),
        grid=(1,),
        in_specs=[
            pl.BlockSpec((T, D), lambda i: (0, 0)),
            pl.BlockSpec((E, D), lambda i: (0, 0)),
            pl.BlockSpec((1, E), lambda i: (0, 0)),
        ],
        out_specs=[
            pl.BlockSpec((T, 1), lambda i: (0, 0)),
            pl.BlockSpec((T, 1), lambda i: (0, 0)),
            pl.BlockSpec((2, R, 128), lambda i: (0, 0, 0)),
            pl.BlockSpec((1, 128), lambda i: (0, 0)),
        ],
        out_shape=[
            jax.ShapeDtypeStruct((T, 1), jnp.float32),
            jax.ShapeDtypeStruct((T, 1), jnp.float32),
            jax.ShapeDtypeStruct((2, R, 128), jnp.int32),
            jax.ShapeDtypeStruct((1, 128), jnp.int32),
        ],
        compiler_params=pltpu.CompilerParams(
            dimension_semantics=("arbitrary",),
        ),
    )(x, Wg, bg.reshape(1, E))
    w1c, w2c, pos, te = route
    n_chunk = n_pairs // _N_WORKERS
    pos3 = pos.reshape(_N_WORKERS, n_chunk // 64, 64)
    te_flat = te.reshape(128)

    x_sorted = _sc_dispatch(x, pos3, n_rows=n_rows, n_chunk=n_chunk)

    grid_spec = pltpu.PrefetchScalarGridSpec(
        num_scalar_prefetch=1,
        grid=(n_tiles,),
        in_specs=[
            pl.BlockSpec((_TILE, D), lambda i, te_s: (i, 0)),
            pl.BlockSpec((1, H, D), lambda i, te_s: (te_s[i], 0, 0)),
            pl.BlockSpec((1, 1, H), lambda i, te_s: (te_s[i], 0, 0)),
            pl.BlockSpec((1, D, H), lambda i, te_s: (te_s[i], 0, 0)),
            pl.BlockSpec((1, 1, D), lambda i, te_s: (te_s[i], 0, 0)),
        ],
        out_specs=pl.BlockSpec((_TILE, D), lambda i, te_s: (i, 0)),
    )
    buf = pl.pallas_call(
        _mlp_body,
        grid_spec=grid_spec,
        out_shape=jax.ShapeDtypeStruct((n_rows, D), jnp.float32),
        compiler_params=pltpu.CompilerParams(
            dimension_semantics=("arbitrary",),
        ),
    )(te_flat, x_sorted, W1, b1.reshape(E, 1, H), W2, b2.reshape(E, 1, D))

    g = _sc_combine(buf, pos3, n_pairs=n_pairs, n_chunk=n_chunk)

    tile_t = 256
    half = T // tile_t
    out = pl.pallas_call(
        _final_body,
        grid=(T // tile_t,),
        in_specs=[
            pl.BlockSpec((tile_t, D), lambda i: (i, 0)),
            pl.BlockSpec((tile_t, D), lambda i: (i, 0)),
            pl.BlockSpec((tile_t, D), lambda i: (i + half, 0)),
            pl.BlockSpec((tile_t, 1), lambda i: (i, 0)),
            pl.BlockSpec((tile_t, 1), lambda i: (i, 0)),
            pl.BlockSpec((1, D), lambda i: (0, 0)),
            pl.BlockSpec((1, D), lambda i: (0, 0)),
        ],
        out_specs=pl.BlockSpec((tile_t, D), lambda i: (i, 0)),
        out_shape=jax.ShapeDtypeStruct((T, D), jnp.float32),
        compiler_params=pltpu.CompilerParams(
            dimension_semantics=("arbitrary",),
        ),
    )(x, g, g, w1c, w2c, gamma.reshape(1, D), beta.reshape(1, D))
    return out
